# baseline (speedup 1.0000x reference)
import functools

import jax
import jax.numpy as jnp
from jax import lax
from jax.experimental import pallas as pl
from jax.experimental.pallas import tpu as pltpu
from jax.experimental.pallas import tpu_sc as plsc

N = 10000
E = 320000
NPAD = 10240
IN128 = 128

NC = 2
NS = 16
NW = NC * NS
ROWS_PER_TILE = NPAD // NW
EDGES_PER_SC = E // NC
EDGES_PER_TILE = EDGES_PER_SC // NS
EB = 128
N_FULL_BLOCKS = EDGES_PER_TILE // EB
TAIL = EDGES_PER_TILE - N_FULL_BLOCKS * EB
HIST_PER_TILE = NPAD // NS

_mesh = plsc.VectorSubcoreMesh(core_axis_name="c", subcore_axis_name="s")
_sc_params = pltpu.CompilerParams(needs_layout_passes=False)


def _dnn_body(x_ref, m_ref, w1_ref, b1_ref, w2_ref, b2_ref, p1w_ref, p1b_ref,
              p2w_ref, p2b_ref, y_ref, pr1_ref, pr2_ref):
    x = x_ref[...]
    h = jnp.maximum(jnp.dot(x, w1_ref[...],
                            preferred_element_type=jnp.float32) + b1_ref[...], 0.0)
    goid = jnp.maximum(jnp.dot(h, w2_ref[...],
                               preferred_element_type=jnp.float32) + b2_ref[...], 0.0)
    y_ref[...] = jnp.where(m_ref[...] > 0, x[:, :IN128], goid)
    pr1 = jnp.dot(goid, p1w_ref[...],
                  preferred_element_type=jnp.float32) + p1b_ref[...]
    pr1_ref[...] = pr1
    pr2_ref[...] = jnp.dot(pr1, p2w_ref[...],
                           preferred_element_type=jnp.float32) + p2b_ref[...]


def _run_dnn(xp, mask_col, dnn_W1, dnn_b1, dnn_W2, dnn_b2, p1_W, p1_b, p2_W, p2_b):
    blk = 1024
    grid = (NPAD // blk,)
    full = lambda shape: pl.BlockSpec(shape, lambda i: (0,) * len(shape))
    row = lambda w: pl.BlockSpec((blk, w), lambda i: (i, 0))
    return pl.pallas_call(
        _dnn_body,
        grid=grid,
        in_specs=[row(512), row(1),
                  full((512, 1024)), full((1, 1024)),
                  full((1024, 128)), full((1, 128)),
                  full((128, 128)), full((1, 128)),
                  full((128, 128)), full((1, 128))],
        out_specs=[row(128), row(128), row(128)],
        out_shape=[jax.ShapeDtypeStruct((NPAD, 128), jnp.float32)] * 3,
    )(xp, mask_col, dnn_W1, dnn_b1.reshape(1, -1), dnn_W2, dnn_b2.reshape(1, -1),
      p1_W, p1_b.reshape(1, -1), p2_W, p2_b.reshape(1, -1))


def _pos_body(mf_ref, pos_ref):
    m = mf_ref[...]
    a = lax.broadcasted_iota(jnp.int32, (128, 128), 0)
    b = lax.broadcasted_iota(jnp.int32, (128, 128), 1)
    upper = (a <= b).astype(jnp.float32)
    rowcum = jnp.dot(m, upper, preferred_element_type=jnp.float32)
    rowsum = rowcum[:, 127:128]
    r0 = lax.broadcasted_iota(jnp.int32, (80, 80), 0)
    r1 = lax.broadcasted_iota(jnp.int32, (80, 80), 1)
    strict = (r1 < r0).astype(jnp.float32)
    bp = jnp.dot(strict, rowsum, preferred_element_type=jnp.float32)
    incl = rowcum + bp
    total = jnp.sum(m)
    jr = (lax.broadcasted_iota(jnp.int32, (80, 128), 0) * 128 +
          lax.broadcasted_iota(jnp.int32, (80, 128), 1)).astype(jnp.float32)
    posf = jnp.where(m > 0, incl - 1.0, total + jr - incl)
    pos_ref[...] = posf.astype(jnp.int32)


def _run_pos(mask_f):
    return pl.pallas_call(
        _pos_body,
        out_shape=jax.ShapeDtypeStruct((80, 128), jnp.int32),
    )(mask_f)


@functools.partial(
    pl.kernel,
    mesh=_mesh,
    out_type=[jax.ShapeDtypeStruct((NPAD, 128), jnp.float32),
              jax.ShapeDtypeStruct((NW, NPAD), jnp.int32)],
    scratch_types=[
        pltpu.VMEM((EB, 128), jnp.float32),
        pltpu.VMEM((64, 128), jnp.float32),
        pltpu.VMEM((EB,), jnp.int32),
        pltpu.VMEM((64,), jnp.int32),
        pltpu.VMEM((EDGES_PER_TILE,), jnp.int32),
        pltpu.VMEM((NPAD,), jnp.int32),
        pltpu.SemaphoreType.DMA,
    ],
    compiler_params=_sc_params,
)
def _scatter_hist_kernel(y_hbm, pos_hbm, dst_hbm, xt_hbm, histp_hbm,
                         rows_v, rows64_v, idx_v, idx64_v, dst_v, hist_v, sem):
    c = lax.axis_index("c")
    s = lax.axis_index("s")
    w = c * NS + s

    base = w * ROWS_PER_TILE
    for k in range(2):
        off = base + k * EB
        pltpu.sync_copy(y_hbm.at[pl.ds(off, EB)], rows_v)
        pltpu.sync_copy(pos_hbm.at[pl.ds(off, EB)], idx_v)
        pltpu.async_copy(rows_v, xt_hbm.at[idx_v], sem).wait()
    off = base + 2 * EB
    pltpu.sync_copy(y_hbm.at[pl.ds(off, 64)], rows64_v)
    pltpu.sync_copy(pos_hbm.at[pl.ds(off, 64)], idx64_v)
    pltpu.async_copy(rows64_v, xt_hbm.at[idx64_v], sem).wait()

    def zero_hist(i, _):
        hist_v[pl.ds(i * 16, 16)] = jnp.zeros((16,), jnp.int32)
        return _
    lax.fori_loop(0, NPAD // 16, zero_hist, 0)

    pltpu.sync_copy(dst_hbm.at[pl.ds(w * EDGES_PER_TILE, EDGES_PER_TILE)], dst_v)
    ones = jnp.ones((16,), jnp.int32)

    def hist_step(e, _):
        idx = dst_v[pl.ds(e * 16, 16)]
        plsc.addupdate_scatter(hist_v, [idx], ones)
        return _
    lax.fori_loop(0, EDGES_PER_TILE // 16, hist_step, 0)

    pltpu.sync_copy(hist_v, histp_hbm.at[w])


def _xws_body(xt_ref, hist_ref, w_ref, xws_ref, dinv_ref):
    deg = jnp.sum(hist_ref[...].astype(jnp.float32), axis=1, keepdims=True) + 1.0
    dinv = lax.rsqrt(deg)
    z = jnp.dot(xt_ref[...], w_ref[...], preferred_element_type=jnp.float32)
    xws_ref[...] = dinv * z
    dinv_ref[...] = dinv


def _run_xws(xt, hist2, conv_W):
    blk = 1024
    return pl.pallas_call(
        _xws_body,
        grid=(NPAD // blk,),
        in_specs=[pl.BlockSpec((blk, 128), lambda i: (i, 0)),
                  pl.BlockSpec((blk, NW), lambda i: (i, 0)),
                  pl.BlockSpec((128, 128), lambda i: (0, 0))],
        out_specs=[pl.BlockSpec((blk, 128), lambda i: (i, 0)),
                   pl.BlockSpec((blk, 1), lambda i: (i, 0))],
        out_shape=[jax.ShapeDtypeStruct((NPAD, 128), jnp.float32),
                   jax.ShapeDtypeStruct((NPAD, 1), jnp.float32)],
    )(xt, hist2, conv_W)


GK = 2
HALF = NPAD // 2
ACC_ROWS = HALF + 8
EPT_B = E // NS
NBLK_B = EPT_B // EB
NGROUPS_B = NBLK_B // GK
TAIL_B = EPT_B - NBLK_B * EB
ZROWS = HALF // NS


@functools.partial(
    pl.kernel,
    mesh=_mesh,
    out_type=jax.ShapeDtypeStruct((NPAD, 128), jnp.float32),
    scratch_types=[
        [[pltpu.VMEM((EB,), jnp.int32) for _ in range(GK)] for _ in range(2)],
        [[pltpu.VMEM((EB,), jnp.int32) for _ in range(GK)] for _ in range(2)],
        pltpu.VMEM((2, GK * EB, 128), jnp.float32),
        pltpu.VMEM((TAIL_B,), jnp.int32),
        pltpu.VMEM((TAIL_B,), jnp.int32),
        pltpu.VMEM((TAIL_B, 128), jnp.float32),
        pltpu.VMEM_SHARED((ACC_ROWS, 128), jnp.float32),
        pltpu.SemaphoreType.DMA,
        pltpu.SemaphoreType.DMA,
        pltpu.SemaphoreType.DMA,
        pltpu.SemaphoreType.DMA,
        pltpu.SemaphoreType.DMA,
        pltpu.SemaphoreType.DMA,
    ],
    compiler_params=_sc_params,
)
def _edge_pass_kernel(xws_hbm, src_hbm, dst_hbm, agg_hbm,
                      sidx, didx, rows_v, sidx_t, didx_t, rows_t,
                      acc_sh, isem0, isem1, gsem0, gsem1, ssem0, ssem1):
    c = lax.axis_index("c")
    s = lax.axis_index("s")
    cbase = c * HALF

    def zero_row(i, _):
        r = i // 8
        k = i % 8
        rows_v[0, r, pl.ds(k * 16, 16)] = jnp.zeros((16,), jnp.float32)
        return _
    lax.fori_loop(0, GK * EB * 8, zero_row, 0)
    rbase = s * ZROWS
    pltpu.sync_copy(rows_v.at[0],
                    acc_sh.at[pl.ds(rbase, GK * EB)])
    pltpu.sync_copy(rows_v.at[0, pl.ds(0, ZROWS - GK * EB)],
                    acc_sh.at[pl.ds(rbase + GK * EB, ZROWS - GK * EB)])

    @pl.when(s == 0)
    def _zero_trash():
        pltpu.sync_copy(rows_v.at[0, pl.ds(0, 8)],
                        acc_sh.at[pl.ds(HALF, 8)])
    plsc.subcore_barrier()

    ebase = s * EPT_B

    def idx_copy(group, buf, sem):
        descs = []
        for j in range(GK):
            off = ebase + group * (GK * EB) + j * EB
            descs.append(pltpu.async_copy(src_hbm.at[pl.ds(off, EB)],
                                          sidx[buf][j], sem))
            descs.append(pltpu.async_copy(dst_hbm.at[pl.ds(off, EB)],
                                          didx[buf][j], sem))
        return descs

    def remap(buf):
        for j in range(GK):
            for k in range(EB // 16):
                d = didx[buf][j][pl.ds(k * 16, 16)] - cbase
                ok = (d >= 0) & (d < HALF)
                didx[buf][j][pl.ds(k * 16, 16)] = jnp.where(ok, d, HALF)

    def remap_tail():
        for k in range(TAIL_B // 16):
            d = didx_t[pl.ds(k * 16, 16)] - cbase
            ok = (d >= 0) & (d < HALF)
            didx_t[pl.ds(k * 16, 16)] = jnp.where(ok, d, HALF)

    def wait_all(descs):
        for d in descs:
            d.wait()

    def fire_gathers(buf, sem):
        return [pltpu.async_copy(xws_hbm.at[sidx[buf][j]],
                                 rows_v.at[buf, pl.ds(j * EB, EB)], sem)
                for j in range(GK)]

    def fire_scatters(buf, sem):
        return [pltpu.async_copy(rows_v.at[buf, pl.ds(j * EB, EB)],
                                 acc_sh.at[didx[buf][j]], sem, add=True)
                for j in range(GK)]

    wait_all(idx_copy(0, 0, isem0))

    def body(i, _):
        gb = 2 * i + 1
        g0 = fire_gathers(0, gsem0)
        i1 = idx_copy(gb, 1, isem1)
        remap(0)
        wait_all(g0)
        s0 = fire_scatters(0, ssem0)
        wait_all(i1)
        g1 = fire_gathers(1, gsem1)
        remap(1)
        wait_all(s0)
        gnext = jnp.minimum(gb + 1, NGROUPS_B - 1)
        inext = idx_copy(gnext, 0, isem0)
        wait_all(g1)
        s1 = fire_scatters(1, ssem1)
        wait_all(inext)
        wait_all(s1)
        return _
    lax.fori_loop(0, NGROUPS_B // 2, body, 0)

    toff = ebase + NBLK_B * EB
    pltpu.sync_copy(src_hbm.at[pl.ds(toff, TAIL_B)], sidx_t)
    pltpu.sync_copy(dst_hbm.at[pl.ds(toff, TAIL_B)], didx_t)
    remap_tail()
    pltpu.async_copy(xws_hbm.at[sidx_t], rows_t, gsem0).wait()
    pltpu.async_copy(rows_t, acc_sh.at[didx_t], ssem0, add=True).wait()

    plsc.subcore_barrier()

    pltpu.sync_copy(acc_sh.at[pl.ds(rbase, ZROWS)],
                    agg_hbm.at[pl.ds(cbase + rbase, ZROWS)])


def _mid_body(p_ref, xws_ref, dinv_ref, pr_ref, m_ref, b_ref, w_ref,
              out_ref):
    agg = p_ref[...] + xws_ref[...]
    dinv = dinv_ref[...]
    h = jnp.maximum(dinv * agg + b_ref[...], 0.0)
    h = jnp.where(m_ref[...] > 0, h, pr_ref[...])
    out_ref[...] = dinv * jnp.dot(h, w_ref[...],
                                  preferred_element_type=jnp.float32)


def _run_mid(p, xws, dinv, proj, mask_col, conv_b, next_W):
    blk = 1024
    row = lambda w: pl.BlockSpec((blk, w), lambda i: (i, 0))
    return pl.pallas_call(
        _mid_body,
        grid=(NPAD // blk,),
        in_specs=[row(128), row(128), row(1), row(128), row(1),
                  pl.BlockSpec((1, 128), lambda i: (0, 0)),
                  pl.BlockSpec((128, 128), lambda i: (0, 0))],
        out_specs=row(128),
        out_shape=jax.ShapeDtypeStruct((NPAD, 128), jnp.float32),
    )(p, xws, dinv, proj, mask_col, conv_b.reshape(1, -1), next_W)


def _final_body(q_ref, xws_ref, dinv_ref, pr_ref, m_ref, b_ref,
                fcw_ref, fcb_ref, out_ref):
    agg = q_ref[...] + xws_ref[...]
    h = jnp.maximum(dinv_ref[...] * agg + b_ref[...], 0.0)
    h = jnp.where(m_ref[...] > 0, h, pr_ref[...])
    out_ref[...] = jnp.dot(h, fcw_ref[...],
                           preferred_element_type=jnp.float32) + fcb_ref[...]


def _run_final(q, xws, dinv, proj, mask_col, conv_b, fc_W, fc_b):
    blk = 1024
    row = lambda w: pl.BlockSpec((blk, w), lambda i: (i, 0))
    return pl.pallas_call(
        _final_body,
        grid=(NPAD // blk,),
        in_specs=[row(128), row(128), row(1), row(128), row(1),
                  pl.BlockSpec((1, 128), lambda i: (0, 0)),
                  pl.BlockSpec((128, 64), lambda i: (0, 0)),
                  pl.BlockSpec((1, 64), lambda i: (0, 0))],
        out_specs=row(64),
        out_shape=jax.ShapeDtypeStruct((NPAD, 64), jnp.float32),
    )(q, xws, dinv, proj, mask_col, conv_b.reshape(1, -1), fc_W,
      fc_b.reshape(1, -1))


def kernel(x, edge_index, mask, dnn_W1, dnn_b1, dnn_W2, dnn_b2,
           conv1_W, conv1_b, conv2_W, conv2_b, p1_W, p1_b, p2_W, p2_b,
           fc_W, fc_b):
    mask = mask.astype(bool)
    xp = jnp.pad(x, ((0, NPAD - N), (0, 0)))
    mask_col = jnp.pad(mask.astype(jnp.int32), (0, NPAD - N)).reshape(NPAD, 1)
    mask_f = jnp.pad(mask.astype(jnp.float32), (0, NPAD - N)).reshape(80, 128)
    src = edge_index[0].astype(jnp.int32)
    dst = edge_index[1].astype(jnp.int32)

    y, proj1, proj2 = _run_dnn(xp, mask_col, dnn_W1, dnn_b1, dnn_W2, dnn_b2,
                               p1_W, p1_b, p2_W, p2_b)
    pos = _run_pos(mask_f).reshape(NPAD)
    xt, histp = _scatter_hist_kernel(y, pos, dst)
    hist2 = histp.transpose(1, 0)

    xws1, dinv = _run_xws(xt, hist2, conv1_W)
    p = _edge_pass_kernel(xws1, src, dst)
    xws2 = _run_mid(p, xws1, dinv, proj1, mask_col, conv1_b, conv2_W)
    q = _edge_pass_kernel(xws2, src, dst)
    out = _run_final(q, xws2, dinv, proj2, mask_col, conv2_b,
                     fc_W, fc_b)
    return out[:N]


# speedup vs baseline: 1.5711x; 1.5711x over previous
import functools

import jax
import jax.numpy as jnp
from jax import lax
from jax.experimental import pallas as pl
from jax.experimental.pallas import tpu as pltpu
from jax.experimental.pallas import tpu_sc as plsc

N = 10000
E = 320000
NPAD = 10240
IN128 = 128

NC = 2
NS = 16
NW = NC * NS
ROWS_PER_TILE = NPAD // NW
EDGES_PER_SC = E // NC
EDGES_PER_TILE = EDGES_PER_SC // NS
EB = 128
N_FULL_BLOCKS = EDGES_PER_TILE // EB
TAIL = EDGES_PER_TILE - N_FULL_BLOCKS * EB
HIST_PER_TILE = NPAD // NS

_mesh = plsc.VectorSubcoreMesh(core_axis_name="c", subcore_axis_name="s")
_sc_params = pltpu.CompilerParams(needs_layout_passes=False)


def _dnn_body(x_ref, m_ref, w1_ref, b1_ref, w2_ref, b2_ref, p1w_ref, p1b_ref,
              p2w_ref, p2b_ref, y_ref, pr1_ref, pr2_ref):
    x = x_ref[...]
    h = jnp.maximum(jnp.dot(x, w1_ref[...],
                            preferred_element_type=jnp.float32) + b1_ref[...], 0.0)
    goid = jnp.maximum(jnp.dot(h, w2_ref[...],
                               preferred_element_type=jnp.float32) + b2_ref[...], 0.0)
    y_ref[...] = jnp.where(m_ref[...] > 0, x[:, :IN128], goid)
    pr1 = jnp.dot(goid, p1w_ref[...],
                  preferred_element_type=jnp.float32) + p1b_ref[...]
    pr1_ref[...] = pr1
    pr2_ref[...] = jnp.dot(pr1, p2w_ref[...],
                           preferred_element_type=jnp.float32) + p2b_ref[...]


def _run_dnn(xp, mask_col, dnn_W1, dnn_b1, dnn_W2, dnn_b2, p1_W, p1_b, p2_W, p2_b):
    blk = 1024
    grid = (NPAD // blk,)
    full = lambda shape: pl.BlockSpec(shape, lambda i: (0,) * len(shape))
    row = lambda w: pl.BlockSpec((blk, w), lambda i: (i, 0))
    return pl.pallas_call(
        _dnn_body,
        grid=grid,
        in_specs=[row(512), row(1),
                  full((512, 1024)), full((1, 1024)),
                  full((1024, 128)), full((1, 128)),
                  full((128, 128)), full((1, 128)),
                  full((128, 128)), full((1, 128))],
        out_specs=[row(128), row(128), row(128)],
        out_shape=[jax.ShapeDtypeStruct((NPAD, 128), jnp.float32)] * 3,
    )(xp, mask_col, dnn_W1, dnn_b1.reshape(1, -1), dnn_W2, dnn_b2.reshape(1, -1),
      p1_W, p1_b.reshape(1, -1), p2_W, p2_b.reshape(1, -1))


def _pos_body(mf_ref, pos_ref):
    m = mf_ref[...]
    a = lax.broadcasted_iota(jnp.int32, (128, 128), 0)
    b = lax.broadcasted_iota(jnp.int32, (128, 128), 1)
    upper = (a <= b).astype(jnp.float32)
    rowcum = jnp.dot(m, upper, preferred_element_type=jnp.float32)
    rowsum = rowcum[:, 127:128]
    r0 = lax.broadcasted_iota(jnp.int32, (80, 80), 0)
    r1 = lax.broadcasted_iota(jnp.int32, (80, 80), 1)
    strict = (r1 < r0).astype(jnp.float32)
    bp = jnp.dot(strict, rowsum, preferred_element_type=jnp.float32)
    incl = rowcum + bp
    total = jnp.sum(m)
    jr = (lax.broadcasted_iota(jnp.int32, (80, 128), 0) * 128 +
          lax.broadcasted_iota(jnp.int32, (80, 128), 1)).astype(jnp.float32)
    posf = jnp.where(m > 0, incl - 1.0, total + jr - incl)
    pos_ref[...] = posf.astype(jnp.int32)


def _run_pos(mask_f):
    return pl.pallas_call(
        _pos_body,
        out_shape=jax.ShapeDtypeStruct((80, 128), jnp.int32),
    )(mask_f)


@functools.partial(
    pl.kernel,
    mesh=_mesh,
    out_type=[jax.ShapeDtypeStruct((NPAD, 128), jnp.float32),
              jax.ShapeDtypeStruct((NW, NPAD), jnp.int32)],
    scratch_types=[
        pltpu.VMEM((EB, 128), jnp.float32),
        pltpu.VMEM((64, 128), jnp.float32),
        pltpu.VMEM((EB,), jnp.int32),
        pltpu.VMEM((64,), jnp.int32),
        pltpu.VMEM((EDGES_PER_TILE,), jnp.int32),
        pltpu.VMEM((NPAD,), jnp.int32),
        pltpu.SemaphoreType.DMA,
    ],
    compiler_params=_sc_params,
)
def _scatter_hist_kernel(y_hbm, pos_hbm, dst_hbm, xt_hbm, histp_hbm,
                         rows_v, rows64_v, idx_v, idx64_v, dst_v, hist_v, sem):
    c = lax.axis_index("c")
    s = lax.axis_index("s")
    w = c * NS + s

    base = w * ROWS_PER_TILE
    for k in range(2):
        off = base + k * EB
        pltpu.sync_copy(y_hbm.at[pl.ds(off, EB)], rows_v)
        pltpu.sync_copy(pos_hbm.at[pl.ds(off, EB)], idx_v)
        pltpu.async_copy(rows_v, xt_hbm.at[idx_v], sem).wait()
    off = base + 2 * EB
    pltpu.sync_copy(y_hbm.at[pl.ds(off, 64)], rows64_v)
    pltpu.sync_copy(pos_hbm.at[pl.ds(off, 64)], idx64_v)
    pltpu.async_copy(rows64_v, xt_hbm.at[idx64_v], sem).wait()

    def zero_hist(i, _):
        hist_v[pl.ds(i * 16, 16)] = jnp.zeros((16,), jnp.int32)
        return _
    lax.fori_loop(0, NPAD // 16, zero_hist, 0)

    pltpu.sync_copy(dst_hbm.at[pl.ds(w * EDGES_PER_TILE, EDGES_PER_TILE)], dst_v)
    ones = jnp.ones((16,), jnp.int32)

    def hist_step(e, _):
        idx = dst_v[pl.ds(e * 16, 16)]
        plsc.addupdate_scatter(hist_v, [idx], ones)
        return _
    lax.fori_loop(0, EDGES_PER_TILE // 16, hist_step, 0)

    pltpu.sync_copy(hist_v, histp_hbm.at[w])


def _xws_body(xt_ref, hist_ref, w_ref, xws_ref, dinv_ref):
    deg = jnp.sum(hist_ref[...].astype(jnp.float32), axis=1, keepdims=True) + 1.0
    dinv = lax.rsqrt(deg)
    z = jnp.dot(xt_ref[...], w_ref[...], preferred_element_type=jnp.float32)
    xws_ref[...] = dinv * z
    dinv_ref[...] = dinv


def _run_xws(xt, hist2, conv_W):
    blk = 1024
    return pl.pallas_call(
        _xws_body,
        grid=(NPAD // blk,),
        in_specs=[pl.BlockSpec((blk, 128), lambda i: (i, 0)),
                  pl.BlockSpec((blk, NW), lambda i: (i, 0)),
                  pl.BlockSpec((128, 128), lambda i: (0, 0))],
        out_specs=[pl.BlockSpec((blk, 128), lambda i: (i, 0)),
                   pl.BlockSpec((blk, 1), lambda i: (i, 0))],
        out_shape=[jax.ShapeDtypeStruct((NPAD, 128), jnp.float32),
                   jax.ShapeDtypeStruct((NPAD, 1), jnp.float32)],
    )(xt, hist2, conv_W)


@functools.partial(
    pl.kernel,
    mesh=_mesh,
    out_type=jax.ShapeDtypeStruct((NC, NPAD, 128), jnp.float32),
    scratch_types=[
        [pltpu.VMEM((EB,), jnp.int32) for _ in range(2)],
        [pltpu.VMEM((EB,), jnp.int32) for _ in range(2)],
        pltpu.VMEM((2, EB, 128), jnp.float32),
        pltpu.VMEM((TAIL,), jnp.int32),
        pltpu.VMEM((TAIL,), jnp.int32),
        pltpu.VMEM((TAIL, 128), jnp.float32),
        pltpu.VMEM_SHARED((NPAD, 128), jnp.float32),
        pltpu.SemaphoreType.DMA,
        pltpu.SemaphoreType.DMA,
        pltpu.SemaphoreType.DMA,
        pltpu.SemaphoreType.DMA,
        pltpu.SemaphoreType.DMA,
        pltpu.SemaphoreType.DMA,
    ],
    compiler_params=_sc_params,
)
def _edge_pass_kernel(xws_hbm, src_hbm, dst_hbm, part_hbm,
                      sidx, didx, rows_v, sidx_t, didx_t, rows_t,
                      acc_sh, isem0, isem1, gsem0, gsem1, ssem0, ssem1):
    c = lax.axis_index("c")
    s = lax.axis_index("s")
    w = c * NS + s

    def zero_row(i, _):
        r = i // 8
        k = i % 8
        rows_v[0, r, pl.ds(k * 16, 16)] = jnp.zeros((16,), jnp.float32)
        return _
    lax.fori_loop(0, EB * 8, zero_row, 0)
    rbase = s * HIST_PER_TILE
    for k in range(HIST_PER_TILE // EB):
        pltpu.sync_copy(rows_v.at[0], acc_sh.at[pl.ds(rbase + k * EB, EB)])
    plsc.subcore_barrier()

    ebase = w * EDGES_PER_TILE

    def idx_copy(blk, buf, sem):
        off = ebase + blk * EB
        return [pltpu.async_copy(src_hbm.at[pl.ds(off, EB)], sidx[buf], sem),
                pltpu.async_copy(dst_hbm.at[pl.ds(off, EB)], didx[buf], sem)]

    def wait_all(descs):
        for d in descs:
            d.wait()

    def gather(buf, sem):
        return pltpu.async_copy(xws_hbm.at[sidx[buf]], rows_v.at[buf], sem)

    def scatter(buf, sem):
        return pltpu.async_copy(rows_v.at[buf], acc_sh.at[didx[buf]], sem,
                                add=True)

    wait_all(idx_copy(0, 0, isem0))

    def body(i, _):
        g0 = gather(0, gsem0)
        i1 = idx_copy(2 * i + 1, 1, isem1)
        g0.wait()
        s0 = scatter(0, ssem0)
        wait_all(i1)
        g1 = gather(1, gsem1)
        s0.wait()
        bnext = jnp.minimum(2 * i + 2, N_FULL_BLOCKS - 1)
        inext = idx_copy(bnext, 0, isem0)
        g1.wait()
        s1 = scatter(1, ssem1)
        wait_all(inext)
        s1.wait()
        return _
    lax.fori_loop(0, N_FULL_BLOCKS // 2, body, 0)

    toff = ebase + N_FULL_BLOCKS * EB
    pltpu.sync_copy(src_hbm.at[pl.ds(toff, TAIL)], sidx_t)
    pltpu.sync_copy(dst_hbm.at[pl.ds(toff, TAIL)], didx_t)
    pltpu.async_copy(xws_hbm.at[sidx_t], rows_t, gsem0).wait()
    pltpu.async_copy(rows_t, acc_sh.at[didx_t], ssem0, add=True).wait()

    plsc.subcore_barrier()

    pltpu.sync_copy(acc_sh.at[pl.ds(rbase, HIST_PER_TILE)],
                    part_hbm.at[c, pl.ds(rbase, HIST_PER_TILE)])


def _mid_body(p0_ref, p1_ref, xws_ref, dinv_ref, pr_ref, m_ref, b_ref, w_ref,
              out_ref):
    agg = p0_ref[...] + p1_ref[...] + xws_ref[...]
    dinv = dinv_ref[...]
    h = jnp.maximum(dinv * agg + b_ref[...], 0.0)
    h = jnp.where(m_ref[...] > 0, h, pr_ref[...])
    out_ref[...] = dinv * jnp.dot(h, w_ref[...],
                                  preferred_element_type=jnp.float32)


def _run_mid(p0, p1, xws, dinv, proj, mask_col, conv_b, next_W):
    blk = 1024
    row = lambda w: pl.BlockSpec((blk, w), lambda i: (i, 0))
    return pl.pallas_call(
        _mid_body,
        grid=(NPAD // blk,),
        in_specs=[row(128), row(128), row(128), row(1), row(128), row(1),
                  pl.BlockSpec((1, 128), lambda i: (0, 0)),
                  pl.BlockSpec((128, 128), lambda i: (0, 0))],
        out_specs=row(128),
        out_shape=jax.ShapeDtypeStruct((NPAD, 128), jnp.float32),
    )(p0, p1, xws, dinv, proj, mask_col, conv_b.reshape(1, -1), next_W)


def _final_body(q0_ref, q1_ref, xws_ref, dinv_ref, pr_ref, m_ref, b_ref,
                fcw_ref, fcb_ref, out_ref):
    agg = q0_ref[...] + q1_ref[...] + xws_ref[...]
    h = jnp.maximum(dinv_ref[...] * agg + b_ref[...], 0.0)
    h = jnp.where(m_ref[...] > 0, h, pr_ref[...])
    out_ref[...] = jnp.dot(h, fcw_ref[...],
                           preferred_element_type=jnp.float32) + fcb_ref[...]


def _run_final(q0, q1, xws, dinv, proj, mask_col, conv_b, fc_W, fc_b):
    blk = 1024
    row = lambda w: pl.BlockSpec((blk, w), lambda i: (i, 0))
    return pl.pallas_call(
        _final_body,
        grid=(NPAD // blk,),
        in_specs=[row(128), row(128), row(128), row(1), row(128), row(1),
                  pl.BlockSpec((1, 128), lambda i: (0, 0)),
                  pl.BlockSpec((128, 64), lambda i: (0, 0)),
                  pl.BlockSpec((1, 64), lambda i: (0, 0))],
        out_specs=row(64),
        out_shape=jax.ShapeDtypeStruct((NPAD, 64), jnp.float32),
    )(q0, q1, xws, dinv, proj, mask_col, conv_b.reshape(1, -1), fc_W,
      fc_b.reshape(1, -1))


def kernel(x, edge_index, mask, dnn_W1, dnn_b1, dnn_W2, dnn_b2,
           conv1_W, conv1_b, conv2_W, conv2_b, p1_W, p1_b, p2_W, p2_b,
           fc_W, fc_b):
    mask = mask.astype(bool)
    xp = jnp.pad(x, ((0, NPAD - N), (0, 0)))
    mask_col = jnp.pad(mask.astype(jnp.int32), (0, NPAD - N)).reshape(NPAD, 1)
    mask_f = jnp.pad(mask.astype(jnp.float32), (0, NPAD - N)).reshape(80, 128)
    src = edge_index[0].astype(jnp.int32)
    dst = edge_index[1].astype(jnp.int32)

    y, proj1, proj2 = _run_dnn(xp, mask_col, dnn_W1, dnn_b1, dnn_W2, dnn_b2,
                               p1_W, p1_b, p2_W, p2_b)
    pos = _run_pos(mask_f).reshape(NPAD)
    xt, histp = _scatter_hist_kernel(y, pos, dst)
    hist2 = histp.transpose(1, 0)

    xws1, dinv = _run_xws(xt, hist2, conv1_W)
    p = _edge_pass_kernel(xws1, src, dst)
    xws2 = _run_mid(p[0], p[1], xws1, dinv, proj1, mask_col, conv1_b, conv2_W)
    q = _edge_pass_kernel(xws2, src, dst)
    out = _run_final(q[0], q[1], xws2, dinv, proj2, mask_col, conv2_b,
                     fc_W, fc_b)
    return out[:N]


# speedup vs baseline: 1.8656x; 1.1874x over previous
import functools

import jax
import jax.numpy as jnp
from jax import lax
from jax.experimental import pallas as pl
from jax.experimental.pallas import tpu as pltpu
from jax.experimental.pallas import tpu_sc as plsc

N = 10000
E = 320000
NPAD = 10240
IN128 = 128

NC = 2
NS = 16
NW = NC * NS
ROWS_PER_TILE = NPAD // NW
EDGES_PER_SC = E // NC
EDGES_PER_TILE = EDGES_PER_SC // NS
EB = 128
N_FULL_BLOCKS = EDGES_PER_TILE // EB
TAIL = EDGES_PER_TILE - N_FULL_BLOCKS * EB
HIST_PER_TILE = NPAD // NS

_mesh = plsc.VectorSubcoreMesh(core_axis_name="c", subcore_axis_name="s")
_sc_params = pltpu.CompilerParams(needs_layout_passes=False)


def _dnn_body(x_ref, m_ref, w1_ref, b1_ref, w2_ref, b2_ref, p1w_ref, p1b_ref,
              p2w_ref, p2b_ref, y_ref, pr1_ref, pr2_ref):
    x = x_ref[...]
    h = jnp.maximum(jnp.dot(x, w1_ref[...],
                            preferred_element_type=jnp.float32) + b1_ref[...], 0.0)
    goid = jnp.maximum(jnp.dot(h, w2_ref[...],
                               preferred_element_type=jnp.float32) + b2_ref[...], 0.0)
    y_ref[...] = jnp.where(m_ref[...] > 0, x[:, :IN128], goid)
    pr1 = jnp.dot(goid, p1w_ref[...],
                  preferred_element_type=jnp.float32) + p1b_ref[...]
    pr1_ref[...] = pr1
    pr2_ref[...] = jnp.dot(pr1, p2w_ref[...],
                           preferred_element_type=jnp.float32) + p2b_ref[...]


def _run_dnn(xp, mask_col, dnn_W1, dnn_b1, dnn_W2, dnn_b2, p1_W, p1_b, p2_W, p2_b):
    blk = 1024
    grid = (NPAD // blk,)
    full = lambda shape: pl.BlockSpec(shape, lambda i: (0,) * len(shape))
    row = lambda w: pl.BlockSpec((blk, w), lambda i: (i, 0))
    return pl.pallas_call(
        _dnn_body,
        grid=grid,
        in_specs=[row(512), row(1),
                  full((512, 1024)), full((1, 1024)),
                  full((1024, 128)), full((1, 128)),
                  full((128, 128)), full((1, 128)),
                  full((128, 128)), full((1, 128))],
        out_specs=[row(128), row(128), row(128)],
        out_shape=[jax.ShapeDtypeStruct((NPAD, 128), jnp.float32)] * 3,
    )(xp, mask_col, dnn_W1, dnn_b1.reshape(1, -1), dnn_W2, dnn_b2.reshape(1, -1),
      p1_W, p1_b.reshape(1, -1), p2_W, p2_b.reshape(1, -1))


def _pos_body(mf_ref, pos_ref):
    m = mf_ref[...]
    a = lax.broadcasted_iota(jnp.int32, (128, 128), 0)
    b = lax.broadcasted_iota(jnp.int32, (128, 128), 1)
    upper = (a <= b).astype(jnp.float32)
    rowcum = jnp.dot(m, upper, preferred_element_type=jnp.float32)
    rowsum = rowcum[:, 127:128]
    r0 = lax.broadcasted_iota(jnp.int32, (80, 80), 0)
    r1 = lax.broadcasted_iota(jnp.int32, (80, 80), 1)
    strict = (r1 < r0).astype(jnp.float32)
    bp = jnp.dot(strict, rowsum, preferred_element_type=jnp.float32)
    incl = rowcum + bp
    total = jnp.sum(m)
    jr = (lax.broadcasted_iota(jnp.int32, (80, 128), 0) * 128 +
          lax.broadcasted_iota(jnp.int32, (80, 128), 1)).astype(jnp.float32)
    posf = jnp.where(m > 0, incl - 1.0, total + jr - incl)
    pos_ref[...] = posf.astype(jnp.int32)


def _run_pos(mask_f):
    return pl.pallas_call(
        _pos_body,
        out_shape=jax.ShapeDtypeStruct((80, 128), jnp.int32),
    )(mask_f)


@functools.partial(
    pl.kernel,
    mesh=_mesh,
    out_type=[jax.ShapeDtypeStruct((NPAD, 128), jnp.float32),
              jax.ShapeDtypeStruct((NW, NPAD), jnp.int32)],
    scratch_types=[
        pltpu.VMEM((EB, 128), jnp.float32),
        pltpu.VMEM((64, 128), jnp.float32),
        pltpu.VMEM((EB,), jnp.int32),
        pltpu.VMEM((64,), jnp.int32),
        pltpu.VMEM((EDGES_PER_TILE,), jnp.int32),
        pltpu.VMEM((NPAD,), jnp.int32),
        pltpu.SemaphoreType.DMA,
    ],
    compiler_params=_sc_params,
)
def _scatter_hist_kernel(y_hbm, pos_hbm, dst_hbm, xt_hbm, histp_hbm,
                         rows_v, rows64_v, idx_v, idx64_v, dst_v, hist_v, sem):
    c = lax.axis_index("c")
    s = lax.axis_index("s")
    w = c * NS + s

    base = w * ROWS_PER_TILE
    for k in range(2):
        off = base + k * EB
        pltpu.sync_copy(y_hbm.at[pl.ds(off, EB)], rows_v)
        pltpu.sync_copy(pos_hbm.at[pl.ds(off, EB)], idx_v)
        pltpu.async_copy(rows_v, xt_hbm.at[idx_v], sem).wait()
    off = base + 2 * EB
    pltpu.sync_copy(y_hbm.at[pl.ds(off, 64)], rows64_v)
    pltpu.sync_copy(pos_hbm.at[pl.ds(off, 64)], idx64_v)
    pltpu.async_copy(rows64_v, xt_hbm.at[idx64_v], sem).wait()

    def zero_hist(i, _):
        hist_v[pl.ds(i * 16, 16)] = jnp.zeros((16,), jnp.int32)
        return _
    lax.fori_loop(0, NPAD // 16, zero_hist, 0)

    pltpu.sync_copy(dst_hbm.at[pl.ds(w * EDGES_PER_TILE, EDGES_PER_TILE)], dst_v)
    ones = jnp.ones((16,), jnp.int32)

    def hist_step(e, _):
        idx = dst_v[pl.ds(e * 16, 16)]
        plsc.addupdate_scatter(hist_v, [idx], ones)
        return _
    lax.fori_loop(0, EDGES_PER_TILE // 16, hist_step, 0)

    pltpu.sync_copy(hist_v, histp_hbm.at[w])


def _xws_body(xt_ref, hist_ref, w_ref, xws_ref, dinv_ref):
    deg = jnp.sum(hist_ref[...].astype(jnp.float32), axis=1, keepdims=True) + 1.0
    dinv = lax.rsqrt(deg)
    z = jnp.dot(xt_ref[...], w_ref[...], preferred_element_type=jnp.float32)
    xws_ref[...] = dinv * z
    dinv_ref[...] = dinv


def _run_xws(xt, hist2, conv_W):
    blk = 1024
    return pl.pallas_call(
        _xws_body,
        grid=(NPAD // blk,),
        in_specs=[pl.BlockSpec((blk, 128), lambda i: (i, 0)),
                  pl.BlockSpec((blk, NW), lambda i: (i, 0)),
                  pl.BlockSpec((128, 128), lambda i: (0, 0))],
        out_specs=[pl.BlockSpec((blk, 128), lambda i: (i, 0)),
                   pl.BlockSpec((blk, 1), lambda i: (i, 0))],
        out_shape=[jax.ShapeDtypeStruct((NPAD, 128), jnp.float32),
                   jax.ShapeDtypeStruct((NPAD, 1), jnp.float32)],
    )(xt, hist2, conv_W)


EB3 = 104
NBLK3 = EDGES_PER_TILE // EB3
BODIES = NBLK3 // 6


@functools.partial(
    pl.kernel,
    mesh=_mesh,
    out_type=jax.ShapeDtypeStruct((NC, NPAD, 128), jnp.float32),
    scratch_types=[
        [[pltpu.VMEM((EB3,), jnp.int32) for _ in range(2)] for _ in range(3)],
        [[pltpu.VMEM((EB3,), jnp.int32) for _ in range(2)] for _ in range(3)],
        pltpu.VMEM((3, EB3, 128), jnp.float32),
        pltpu.VMEM((TAIL,), jnp.int32),
        pltpu.VMEM((TAIL,), jnp.int32),
        pltpu.VMEM((TAIL, 128), jnp.float32),
        pltpu.VMEM_SHARED((NPAD, 128), jnp.float32),
        [[pltpu.SemaphoreType.DMA for _ in range(2)] for _ in range(3)],
        [pltpu.SemaphoreType.DMA for _ in range(3)],
        [pltpu.SemaphoreType.DMA for _ in range(3)],
    ],
    compiler_params=_sc_params,
)
def _edge_pass_kernel(xws_hbm, src_hbm, dst_hbm, part_hbm,
                      sidx, didx, rows_v, sidx_t, didx_t, rows_t,
                      acc_sh, isem, gsem, ssem):
    c = lax.axis_index("c")
    s = lax.axis_index("s")
    w = c * NS + s

    def zero_row(i, _):
        r = i // 8
        k = i % 8
        rows_v[0, r, pl.ds(k * 16, 16)] = jnp.zeros((16,), jnp.float32)
        return _
    lax.fori_loop(0, EB3 * 8, zero_row, 0)
    rbase = s * HIST_PER_TILE
    for k in range(HIST_PER_TILE // EB3):
        pltpu.sync_copy(rows_v.at[0], acc_sh.at[pl.ds(rbase + k * EB3, EB3)])
    pltpu.sync_copy(rows_v.at[0, pl.ds(0, HIST_PER_TILE % EB3)],
                    acc_sh.at[pl.ds(rbase + (HIST_PER_TILE // EB3) * EB3,
                                    HIST_PER_TILE % EB3)])
    plsc.subcore_barrier()

    ebase = w * EDGES_PER_TILE

    def stage_idx(blk, slot, par):
        off = ebase + blk * EB3
        pltpu.async_copy(src_hbm.at[pl.ds(off, EB3)], sidx[slot][par],
                         isem[slot][par])
        pltpu.async_copy(dst_hbm.at[pl.ds(off, EB3)], didx[slot][par],
                         isem[slot][par])

    def iwait(slot, par):
        pltpu.make_async_copy(src_hbm.at[pl.ds(0, EB3)], sidx[slot][par],
                              isem[slot][par]).wait()
        pltpu.make_async_copy(dst_hbm.at[pl.ds(0, EB3)], didx[slot][par],
                              isem[slot][par]).wait()

    def swait(slot):
        pltpu.make_async_copy(xws_hbm.at[pl.ds(0, EB3)], rows_v.at[slot],
                              ssem[slot]).wait()

    def fire_scatter(slot, par):
        pltpu.async_copy(rows_v.at[slot], acc_sh.at[didx[slot][par]],
                         ssem[slot], add=True)

    for slot in range(3):
        stage_idx(slot, slot, 0)

    def body(i, _):
        g = [None] * 3
        for j in range(6):
            slot = j % 3
            par = 0 if j < 3 else 1
            if j < 3:
                @pl.when(i > 0)
                def _drain(slot=slot):
                    swait(slot)
            else:
                swait(slot)
            stage_idx(jnp.minimum(6 * i + j + 3, NBLK3 - 1), slot, 1 - par)
            iwait(slot, par)
            g[slot] = pltpu.async_copy(xws_hbm.at[sidx[slot][par]],
                                       rows_v.at[slot], gsem[slot])
            if j > 0:
                prev = (j - 1) % 3
                g[prev].wait()
                fire_scatter(prev, 0 if j - 1 < 3 else 1)
        g[2].wait()
        fire_scatter(2, 1)
        return _
    lax.fori_loop(0, BODIES, body, 0)

    for slot in range(3):
        swait(slot)
        iwait(slot, 0)

    toff = ebase + NBLK3 * EB3
    pltpu.sync_copy(src_hbm.at[pl.ds(toff, TAIL)], sidx_t)
    pltpu.sync_copy(dst_hbm.at[pl.ds(toff, TAIL)], didx_t)
    pltpu.async_copy(xws_hbm.at[sidx_t], rows_t, gsem[0]).wait()
    pltpu.async_copy(rows_t, acc_sh.at[didx_t], ssem[0], add=True).wait()

    plsc.subcore_barrier()

    pltpu.sync_copy(acc_sh.at[pl.ds(rbase, HIST_PER_TILE)],
                    part_hbm.at[c, pl.ds(rbase, HIST_PER_TILE)])


def _mid_body(p0_ref, p1_ref, xws_ref, dinv_ref, pr_ref, m_ref, b_ref, w_ref,
              out_ref):
    agg = p0_ref[...] + p1_ref[...] + xws_ref[...]
    dinv = dinv_ref[...]
    h = jnp.maximum(dinv * agg + b_ref[...], 0.0)
    h = jnp.where(m_ref[...] > 0, h, pr_ref[...])
    out_ref[...] = dinv * jnp.dot(h, w_ref[...],
                                  preferred_element_type=jnp.float32)


def _run_mid(p0, p1, xws, dinv, proj, mask_col, conv_b, next_W):
    blk = 1024
    row = lambda w: pl.BlockSpec((blk, w), lambda i: (i, 0))
    return pl.pallas_call(
        _mid_body,
        grid=(NPAD // blk,),
        in_specs=[row(128), row(128), row(128), row(1), row(128), row(1),
                  pl.BlockSpec((1, 128), lambda i: (0, 0)),
                  pl.BlockSpec((128, 128), lambda i: (0, 0))],
        out_specs=row(128),
        out_shape=jax.ShapeDtypeStruct((NPAD, 128), jnp.float32),
    )(p0, p1, xws, dinv, proj, mask_col, conv_b.reshape(1, -1), next_W)


def _final_body(q0_ref, q1_ref, xws_ref, dinv_ref, pr_ref, m_ref, b_ref,
                fcw_ref, fcb_ref, out_ref):
    agg = q0_ref[...] + q1_ref[...] + xws_ref[...]
    h = jnp.maximum(dinv_ref[...] * agg + b_ref[...], 0.0)
    h = jnp.where(m_ref[...] > 0, h, pr_ref[...])
    out_ref[...] = jnp.dot(h, fcw_ref[...],
                           preferred_element_type=jnp.float32) + fcb_ref[...]


def _run_final(q0, q1, xws, dinv, proj, mask_col, conv_b, fc_W, fc_b):
    blk = 1024
    row = lambda w: pl.BlockSpec((blk, w), lambda i: (i, 0))
    return pl.pallas_call(
        _final_body,
        grid=(NPAD // blk,),
        in_specs=[row(128), row(128), row(128), row(1), row(128), row(1),
                  pl.BlockSpec((1, 128), lambda i: (0, 0)),
                  pl.BlockSpec((128, 64), lambda i: (0, 0)),
                  pl.BlockSpec((1, 64), lambda i: (0, 0))],
        out_specs=row(64),
        out_shape=jax.ShapeDtypeStruct((NPAD, 64), jnp.float32),
    )(q0, q1, xws, dinv, proj, mask_col, conv_b.reshape(1, -1), fc_W,
      fc_b.reshape(1, -1))


def kernel(x, edge_index, mask, dnn_W1, dnn_b1, dnn_W2, dnn_b2,
           conv1_W, conv1_b, conv2_W, conv2_b, p1_W, p1_b, p2_W, p2_b,
           fc_W, fc_b):
    mask = mask.astype(bool)
    xp = jnp.pad(x, ((0, NPAD - N), (0, 0)))
    mask_col = jnp.pad(mask.astype(jnp.int32), (0, NPAD - N)).reshape(NPAD, 1)
    mask_f = jnp.pad(mask.astype(jnp.float32), (0, NPAD - N)).reshape(80, 128)
    src = edge_index[0].astype(jnp.int32)
    dst = edge_index[1].astype(jnp.int32)

    y, proj1, proj2 = _run_dnn(xp, mask_col, dnn_W1, dnn_b1, dnn_W2, dnn_b2,
                               p1_W, p1_b, p2_W, p2_b)
    pos = _run_pos(mask_f).reshape(NPAD)
    xt, histp = _scatter_hist_kernel(y, pos, dst)
    hist2 = histp.transpose(1, 0)

    xws1, dinv = _run_xws(xt, hist2, conv1_W)
    p = _edge_pass_kernel(xws1, src, dst)
    xws2 = _run_mid(p[0], p[1], xws1, dinv, proj1, mask_col, conv1_b, conv2_W)
    q = _edge_pass_kernel(xws2, src, dst)
    out = _run_final(q[0], q[1], xws2, dinv, proj2, mask_col, conv2_b,
                     fc_W, fc_b)
    return out[:N]


# speedup vs baseline: 1.9207x; 1.0295x over previous
import functools

import jax
import jax.numpy as jnp
from jax import lax
from jax.experimental import pallas as pl
from jax.experimental.pallas import tpu as pltpu
from jax.experimental.pallas import tpu_sc as plsc

N = 10000
E = 320000
NPAD = 10240
IN128 = 128

NC = 2
NS = 16
NW = NC * NS
ROWS_PER_TILE = NPAD // NW
EDGES_PER_SC = E // NC
EDGES_PER_TILE = EDGES_PER_SC // NS
EB = 128
N_FULL_BLOCKS = EDGES_PER_TILE // EB
TAIL = EDGES_PER_TILE - N_FULL_BLOCKS * EB
HIST_PER_TILE = NPAD // NS

_mesh = plsc.VectorSubcoreMesh(core_axis_name="c", subcore_axis_name="s")
_sc_params = pltpu.CompilerParams(needs_layout_passes=False)


def _dnn_body(x_ref, m_ref, mf_ref, w1_ref, b1_ref, w2_ref, b2_ref, p1w_ref,
              p1b_ref, p2w_ref, p2b_ref, y_ref, pr1_ref, pr2_ref, pos_ref):
    x = x_ref[...]
    h = jnp.maximum(jnp.dot(x, w1_ref[...],
                            preferred_element_type=jnp.float32) + b1_ref[...], 0.0)
    goid = jnp.maximum(jnp.dot(h, w2_ref[...],
                               preferred_element_type=jnp.float32) + b2_ref[...], 0.0)
    y_ref[...] = jnp.where(m_ref[...] > 0, x[:, :IN128], goid)
    pr1 = jnp.dot(goid, p1w_ref[...],
                  preferred_element_type=jnp.float32) + p1b_ref[...]
    pr1_ref[...] = pr1
    pr2_ref[...] = jnp.dot(pr1, p2w_ref[...],
                           preferred_element_type=jnp.float32) + p2b_ref[...]

    @pl.when(pl.program_id(0) == 0)
    def _pos():
        m = mf_ref[...]
        a = lax.broadcasted_iota(jnp.int32, (128, 128), 0)
        b = lax.broadcasted_iota(jnp.int32, (128, 128), 1)
        upper = (a <= b).astype(jnp.float32)
        rowcum = jnp.dot(m, upper, preferred_element_type=jnp.float32)
        rowsum = rowcum[:, 127:128]
        r0 = lax.broadcasted_iota(jnp.int32, (80, 80), 0)
        r1 = lax.broadcasted_iota(jnp.int32, (80, 80), 1)
        strict = (r1 < r0).astype(jnp.float32)
        bp = jnp.dot(strict, rowsum, preferred_element_type=jnp.float32)
        incl = rowcum + bp
        total = jnp.sum(m)
        jr = (lax.broadcasted_iota(jnp.int32, (80, 128), 0) * 128 +
              lax.broadcasted_iota(jnp.int32, (80, 128), 1)).astype(jnp.float32)
        posf = jnp.where(m > 0, incl - 1.0, total + jr - incl)
        pos_ref[...] = posf.astype(jnp.int32)


def _run_dnn(x, mask_col, mask_f, dnn_W1, dnn_b1, dnn_W2, dnn_b2,
             p1_W, p1_b, p2_W, p2_b):
    blk = 1000
    grid = (N // blk,)
    full = lambda shape: pl.BlockSpec(shape, lambda i: (0,) * len(shape))
    row = lambda w: pl.BlockSpec((blk, w), lambda i: (i, 0))
    return pl.pallas_call(
        _dnn_body,
        grid=grid,
        in_specs=[row(512), row(1), full((80, 128)),
                  full((512, 1024)), full((1, 1024)),
                  full((1024, 128)), full((1, 128)),
                  full((128, 128)), full((1, 128)),
                  full((128, 128)), full((1, 128))],
        out_specs=[row(128), row(128), row(128), full((80, 128))],
        out_shape=[jax.ShapeDtypeStruct((N, 128), jnp.float32)] * 3 +
                  [jax.ShapeDtypeStruct((80, 128), jnp.int32)],
    )(x, mask_col, mask_f, dnn_W1, dnn_b1.reshape(1, -1), dnn_W2,
      dnn_b2.reshape(1, -1), p1_W, p1_b.reshape(1, -1), p2_W,
      p2_b.reshape(1, -1))


@functools.partial(
    pl.kernel,
    mesh=_mesh,
    out_type=[jax.ShapeDtypeStruct((NPAD, 128), jnp.float32),
              jax.ShapeDtypeStruct((NW, NPAD), jnp.int32)],
    scratch_types=[
        pltpu.VMEM((EB, 128), jnp.float32),
        pltpu.VMEM((64, 128), jnp.float32),
        pltpu.VMEM((16, 128), jnp.float32),
        pltpu.VMEM((EB,), jnp.int32),
        pltpu.VMEM((64,), jnp.int32),
        pltpu.VMEM((16,), jnp.int32),
        pltpu.VMEM((EDGES_PER_TILE,), jnp.int32),
        pltpu.VMEM((NPAD,), jnp.int32),
        pltpu.SemaphoreType.DMA,
    ],
    compiler_params=_sc_params,
)
def _scatter_hist_kernel(y_hbm, pos_hbm, dst_hbm, xt_hbm, histp_hbm,
                         rows_v, rows64_v, rows16_v, idx_v, idx64_v, idx16_v,
                         dst_v, hist_v, sem):
    c = lax.axis_index("c")
    s = lax.axis_index("s")
    w = c * NS + s

    base = w * ROWS_PER_TILE

    @pl.when(w < NW - 1)
    def _full_tile():
        for k in range(2):
            off = base + k * EB
            pltpu.sync_copy(y_hbm.at[pl.ds(off, EB)], rows_v)
            pltpu.sync_copy(pos_hbm.at[pl.ds(off, EB)], idx_v)
            pltpu.async_copy(rows_v, xt_hbm.at[idx_v], sem).wait()
        off = base + 2 * EB
        pltpu.sync_copy(y_hbm.at[pl.ds(off, 64)], rows64_v)
        pltpu.sync_copy(pos_hbm.at[pl.ds(off, 64)], idx64_v)
        pltpu.async_copy(rows64_v, xt_hbm.at[idx64_v], sem).wait()

    @pl.when(w == NW - 1)
    def _last_tile():
        pltpu.sync_copy(y_hbm.at[pl.ds(base, 64)], rows64_v)
        pltpu.sync_copy(pos_hbm.at[pl.ds(base, 64)], idx64_v)
        pltpu.async_copy(rows64_v, xt_hbm.at[idx64_v], sem).wait()
        pltpu.sync_copy(y_hbm.at[pl.ds(base + 64, 16)], rows16_v)
        pltpu.sync_copy(pos_hbm.at[pl.ds(base + 64, 16)], idx16_v)
        pltpu.async_copy(rows16_v, xt_hbm.at[idx16_v], sem).wait()

    def zero_hist(i, _):
        hist_v[pl.ds(i * 16, 16)] = jnp.zeros((16,), jnp.int32)
        return _
    lax.fori_loop(0, NPAD // 16, zero_hist, 0)

    pltpu.sync_copy(dst_hbm.at[pl.ds(w * EDGES_PER_TILE, EDGES_PER_TILE)], dst_v)
    ones = jnp.ones((16,), jnp.int32)

    def hist_step(e, _):
        idx = dst_v[pl.ds(e * 16, 16)]
        plsc.addupdate_scatter(hist_v, [idx], ones)
        return _
    lax.fori_loop(0, EDGES_PER_TILE // 16, hist_step, 0)

    pltpu.sync_copy(hist_v, histp_hbm.at[w])


def _xws_body(xt_ref, hist_ref, w_ref, xws_ref, dinv_ref):
    deg = jnp.sum(hist_ref[...].astype(jnp.float32), axis=1, keepdims=True) + 1.0
    dinv = lax.rsqrt(deg)
    z = jnp.dot(xt_ref[...], w_ref[...], preferred_element_type=jnp.float32)
    xws_ref[...] = dinv * z
    dinv_ref[...] = dinv


def _run_xws(xt, hist2, conv_W):
    blk = 1000
    return pl.pallas_call(
        _xws_body,
        grid=(N // blk,),
        in_specs=[pl.BlockSpec((blk, 128), lambda i: (i, 0)),
                  pl.BlockSpec((blk, NW), lambda i: (i, 0)),
                  pl.BlockSpec((128, 128), lambda i: (0, 0))],
        out_specs=[pl.BlockSpec((blk, 128), lambda i: (i, 0)),
                   pl.BlockSpec((blk, 1), lambda i: (i, 0))],
        out_shape=[jax.ShapeDtypeStruct((N, 128), jnp.float32),
                   jax.ShapeDtypeStruct((N, 1), jnp.float32)],
    )(xt, hist2, conv_W)


EB3 = 104
NBLK3 = EDGES_PER_TILE // EB3
BODIES = NBLK3 // 6


@functools.partial(
    pl.kernel,
    mesh=_mesh,
    out_type=jax.ShapeDtypeStruct((NC, NPAD, 128), jnp.float32),
    scratch_types=[
        [[pltpu.VMEM((EB3,), jnp.int32) for _ in range(2)] for _ in range(3)],
        [[pltpu.VMEM((EB3,), jnp.int32) for _ in range(2)] for _ in range(3)],
        pltpu.VMEM((3, EB3, 128), jnp.float32),
        pltpu.VMEM((TAIL,), jnp.int32),
        pltpu.VMEM((TAIL,), jnp.int32),
        pltpu.VMEM((TAIL, 128), jnp.float32),
        pltpu.VMEM_SHARED((NPAD, 128), jnp.float32),
        [[pltpu.SemaphoreType.DMA for _ in range(2)] for _ in range(3)],
        [pltpu.SemaphoreType.DMA for _ in range(3)],
        [pltpu.SemaphoreType.DMA for _ in range(3)],
    ],
    compiler_params=_sc_params,
)
def _edge_pass_kernel(xws_hbm, src_hbm, dst_hbm, part_hbm,
                      sidx, didx, rows_v, sidx_t, didx_t, rows_t,
                      acc_sh, isem, gsem, ssem):
    c = lax.axis_index("c")
    s = lax.axis_index("s")
    w = c * NS + s

    def zero_row(i, _):
        r = i // 8
        k = i % 8
        rows_v[0, r, pl.ds(k * 16, 16)] = jnp.zeros((16,), jnp.float32)
        return _
    lax.fori_loop(0, EB3 * 8, zero_row, 0)
    rbase = s * HIST_PER_TILE
    for k in range(HIST_PER_TILE // EB3):
        pltpu.sync_copy(rows_v.at[0], acc_sh.at[pl.ds(rbase + k * EB3, EB3)])
    pltpu.sync_copy(rows_v.at[0, pl.ds(0, HIST_PER_TILE % EB3)],
                    acc_sh.at[pl.ds(rbase + (HIST_PER_TILE // EB3) * EB3,
                                    HIST_PER_TILE % EB3)])
    plsc.subcore_barrier()

    ebase = w * EDGES_PER_TILE

    def stage_idx(blk, slot, par):
        off = ebase + blk * EB3
        pltpu.async_copy(src_hbm.at[pl.ds(off, EB3)], sidx[slot][par],
                         isem[slot][par])
        pltpu.async_copy(dst_hbm.at[pl.ds(off, EB3)], didx[slot][par],
                         isem[slot][par])

    def iwait(slot, par):
        pltpu.make_async_copy(src_hbm.at[pl.ds(0, EB3)], sidx[slot][par],
                              isem[slot][par]).wait()
        pltpu.make_async_copy(dst_hbm.at[pl.ds(0, EB3)], didx[slot][par],
                              isem[slot][par]).wait()

    def swait(slot):
        pltpu.make_async_copy(xws_hbm.at[pl.ds(0, EB3)], rows_v.at[slot],
                              ssem[slot]).wait()

    def fire_scatter(slot, par):
        pltpu.async_copy(rows_v.at[slot], acc_sh.at[didx[slot][par]],
                         ssem[slot], add=True)

    for slot in range(3):
        stage_idx(slot, slot, 0)

    def body(i, _):
        g = [None] * 3
        for j in range(6):
            slot = j % 3
            par = 0 if j < 3 else 1
            if j < 3:
                @pl.when(i > 0)
                def _drain(slot=slot):
                    swait(slot)
            else:
                swait(slot)
            stage_idx(jnp.minimum(6 * i + j + 3, NBLK3 - 1), slot, 1 - par)
            iwait(slot, par)
            g[slot] = pltpu.async_copy(xws_hbm.at[sidx[slot][par]],
                                       rows_v.at[slot], gsem[slot])
            if j > 0:
                prev = (j - 1) % 3
                g[prev].wait()
                fire_scatter(prev, 0 if j - 1 < 3 else 1)
        g[2].wait()
        fire_scatter(2, 1)
        return _
    lax.fori_loop(0, BODIES, body, 0)

    for slot in range(3):
        swait(slot)
        iwait(slot, 0)

    toff = ebase + NBLK3 * EB3
    pltpu.sync_copy(src_hbm.at[pl.ds(toff, TAIL)], sidx_t)
    pltpu.sync_copy(dst_hbm.at[pl.ds(toff, TAIL)], didx_t)
    pltpu.async_copy(xws_hbm.at[sidx_t], rows_t, gsem[0]).wait()
    pltpu.async_copy(rows_t, acc_sh.at[didx_t], ssem[0], add=True).wait()

    plsc.subcore_barrier()

    pltpu.sync_copy(acc_sh.at[pl.ds(rbase, HIST_PER_TILE)],
                    part_hbm.at[c, pl.ds(rbase, HIST_PER_TILE)])


def _mid_body(p0_ref, p1_ref, xws_ref, dinv_ref, pr_ref, m_ref, b_ref, w_ref,
              out_ref):
    agg = p0_ref[...] + p1_ref[...] + xws_ref[...]
    dinv = dinv_ref[...]
    h = jnp.maximum(dinv * agg + b_ref[...], 0.0)
    h = jnp.where(m_ref[...] > 0, h, pr_ref[...])
    out_ref[...] = dinv * jnp.dot(h, w_ref[...],
                                  preferred_element_type=jnp.float32)


def _run_mid(p0, p1, xws, dinv, proj, mask_col, conv_b, next_W):
    blk = 1000
    row = lambda w: pl.BlockSpec((blk, w), lambda i: (i, 0))
    return pl.pallas_call(
        _mid_body,
        grid=(N // blk,),
        in_specs=[row(128), row(128), row(128), row(1), row(128), row(1),
                  pl.BlockSpec((1, 128), lambda i: (0, 0)),
                  pl.BlockSpec((128, 128), lambda i: (0, 0))],
        out_specs=row(128),
        out_shape=jax.ShapeDtypeStruct((N, 128), jnp.float32),
    )(p0, p1, xws, dinv, proj, mask_col, conv_b.reshape(1, -1), next_W)


def _final_body(q0_ref, q1_ref, xws_ref, dinv_ref, pr_ref, m_ref, b_ref,
                fcw_ref, fcb_ref, out_ref):
    agg = q0_ref[...] + q1_ref[...] + xws_ref[...]
    h = jnp.maximum(dinv_ref[...] * agg + b_ref[...], 0.0)
    h = jnp.where(m_ref[...] > 0, h, pr_ref[...])
    out_ref[...] = jnp.dot(h, fcw_ref[...],
                           preferred_element_type=jnp.float32) + fcb_ref[...]


def _run_final(q0, q1, xws, dinv, proj, mask_col, conv_b, fc_W, fc_b):
    blk = 1000
    row = lambda w: pl.BlockSpec((blk, w), lambda i: (i, 0))
    return pl.pallas_call(
        _final_body,
        grid=(N // blk,),
        in_specs=[row(128), row(128), row(128), row(1), row(128), row(1),
                  pl.BlockSpec((1, 128), lambda i: (0, 0)),
                  pl.BlockSpec((128, 64), lambda i: (0, 0)),
                  pl.BlockSpec((1, 64), lambda i: (0, 0))],
        out_specs=row(64),
        out_shape=jax.ShapeDtypeStruct((N, 64), jnp.float32),
    )(q0, q1, xws, dinv, proj, mask_col, conv_b.reshape(1, -1), fc_W,
      fc_b.reshape(1, -1))


def kernel(x, edge_index, mask, dnn_W1, dnn_b1, dnn_W2, dnn_b2,
           conv1_W, conv1_b, conv2_W, conv2_b, p1_W, p1_b, p2_W, p2_b,
           fc_W, fc_b):
    mask = mask.astype(bool)
    mask_col = mask.astype(jnp.int32).reshape(N, 1)
    mask_f = jnp.pad(mask.astype(jnp.float32), (0, NPAD - N)).reshape(80, 128)
    src = edge_index[0].astype(jnp.int32)
    dst = edge_index[1].astype(jnp.int32)

    y, proj1, proj2, pos = _run_dnn(x, mask_col, mask_f, dnn_W1, dnn_b1,
                                    dnn_W2, dnn_b2, p1_W, p1_b, p2_W, p2_b)
    xt, histp = _scatter_hist_kernel(y, pos.reshape(NPAD), dst)
    hist2 = histp.transpose(1, 0)

    xws1, dinv = _run_xws(xt, hist2, conv1_W)
    p = _edge_pass_kernel(xws1, src, dst)
    xws2 = _run_mid(p[0], p[1], xws1, dinv, proj1, mask_col, conv1_b, conv2_W)
    q = _edge_pass_kernel(xws2, src, dst)
    out = _run_final(q[0], q[1], xws2, dinv, proj2, mask_col, conv2_b,
                     fc_W, fc_b)
    return out


# speedup vs baseline: 1.9697x; 1.0255x over previous
import functools

import jax
import jax.numpy as jnp
from jax import lax
from jax.experimental import pallas as pl
from jax.experimental.pallas import tpu as pltpu
from jax.experimental.pallas import tpu_sc as plsc

N = 10000
E = 320000
NPAD = 10240
IN128 = 128

NC = 2
NS = 16
NW = NC * NS
ROWS_PER_TILE = NPAD // NW
EDGES_PER_SC = E // NC
EDGES_PER_TILE = EDGES_PER_SC // NS
EB = 128
N_FULL_BLOCKS = EDGES_PER_TILE // EB
TAIL = EDGES_PER_TILE - N_FULL_BLOCKS * EB
HIST_PER_TILE = NPAD // NS

_mesh = plsc.VectorSubcoreMesh(core_axis_name="c", subcore_axis_name="s")
_sc_params = pltpu.CompilerParams(needs_layout_passes=False)


def _dnn_body(x_ref, m_ref, mf_ref, w1_ref, b1_ref, w2_ref, b2_ref, p1w_ref,
              p1b_ref, p2w_ref, p2b_ref, y_ref, pr1_ref, pr2_ref, pos_ref):
    x = x_ref[...]
    h = jnp.maximum(jnp.dot(x, w1_ref[...],
                            preferred_element_type=jnp.float32) + b1_ref[...], 0.0)
    goid = jnp.maximum(jnp.dot(h, w2_ref[...],
                               preferred_element_type=jnp.float32) + b2_ref[...], 0.0)
    y_ref[...] = jnp.where(m_ref[...] > 0, x[:, :IN128], goid)
    pr1 = jnp.dot(goid, p1w_ref[...],
                  preferred_element_type=jnp.float32) + p1b_ref[...]
    pr1_ref[...] = pr1
    pr2_ref[...] = jnp.dot(pr1, p2w_ref[...],
                           preferred_element_type=jnp.float32) + p2b_ref[...]

    @pl.when(pl.program_id(0) == 0)
    def _pos():
        m = mf_ref[...]
        a = lax.broadcasted_iota(jnp.int32, (128, 128), 0)
        b = lax.broadcasted_iota(jnp.int32, (128, 128), 1)
        upper = (a <= b).astype(jnp.float32)
        rowcum = jnp.dot(m, upper, preferred_element_type=jnp.float32)
        rowsum = rowcum[:, 127:128]
        r0 = lax.broadcasted_iota(jnp.int32, (80, 80), 0)
        r1 = lax.broadcasted_iota(jnp.int32, (80, 80), 1)
        strict = (r1 < r0).astype(jnp.float32)
        bp = jnp.dot(strict, rowsum, preferred_element_type=jnp.float32)
        incl = rowcum + bp
        total = jnp.sum(m)
        jr = (lax.broadcasted_iota(jnp.int32, (80, 128), 0) * 128 +
              lax.broadcasted_iota(jnp.int32, (80, 128), 1)).astype(jnp.float32)
        posf = jnp.where(m > 0, incl - 1.0, total + jr - incl)
        pos_ref[...] = posf.astype(jnp.int32)


def _run_dnn(x, mask_col, mask_f, dnn_W1, dnn_b1, dnn_W2, dnn_b2,
             p1_W, p1_b, p2_W, p2_b):
    blk = 1000
    grid = (N // blk,)
    full = lambda shape: pl.BlockSpec(shape, lambda i: (0,) * len(shape))
    row = lambda w: pl.BlockSpec((blk, w), lambda i: (i, 0))
    return pl.pallas_call(
        _dnn_body,
        grid=grid,
        in_specs=[row(512), row(1), full((80, 128)),
                  full((512, 1024)), full((1, 1024)),
                  full((1024, 128)), full((1, 128)),
                  full((128, 128)), full((1, 128)),
                  full((128, 128)), full((1, 128))],
        out_specs=[row(128), row(128), row(128), full((80, 128))],
        out_shape=[jax.ShapeDtypeStruct((N, 128), jnp.float32)] * 3 +
                  [jax.ShapeDtypeStruct((80, 128), jnp.int32)],
    )(x, mask_col, mask_f, dnn_W1, dnn_b1.reshape(1, -1), dnn_W2,
      dnn_b2.reshape(1, -1), p1_W, p1_b.reshape(1, -1), p2_W,
      p2_b.reshape(1, -1))


@functools.partial(
    pl.kernel,
    mesh=_mesh,
    out_type=jax.ShapeDtypeStruct((NW, NPAD), jnp.int32),
    scratch_types=[
        pltpu.VMEM((EDGES_PER_TILE,), jnp.int32),
        pltpu.VMEM((NPAD,), jnp.int32),
    ],
    compiler_params=_sc_params,
)
def _hist_kernel(dst_hbm, histp_hbm, dst_v, hist_v):
    c = lax.axis_index("c")
    s = lax.axis_index("s")
    w = c * NS + s

    def zero_hist(i, _):
        hist_v[pl.ds(i * 16, 16)] = jnp.zeros((16,), jnp.int32)
        return _
    lax.fori_loop(0, NPAD // 16, zero_hist, 0)

    pltpu.sync_copy(dst_hbm.at[pl.ds(w * EDGES_PER_TILE, EDGES_PER_TILE)], dst_v)
    ones = jnp.ones((16,), jnp.int32)

    def hist_step(e, _):
        idx = dst_v[pl.ds(e * 16, 16)]
        plsc.addupdate_scatter(hist_v, [idx], ones)
        return _
    lax.fori_loop(0, EDGES_PER_TILE // 16, hist_step, 0)

    pltpu.sync_copy(hist_v, histp_hbm.at[w])


@functools.partial(
    pl.kernel,
    mesh=_mesh,
    out_type=jax.ShapeDtypeStruct((NPAD, 128), jnp.float32),
    scratch_types=[
        pltpu.VMEM((EB, 128), jnp.float32),
        pltpu.VMEM((64, 128), jnp.float32),
        pltpu.VMEM((16, 128), jnp.float32),
        pltpu.VMEM((EB,), jnp.int32),
        pltpu.VMEM((64,), jnp.int32),
        pltpu.VMEM((16,), jnp.int32),
        pltpu.SemaphoreType.DMA,
    ],
    compiler_params=_sc_params,
)
def _xt_scatter_kernel(y_hbm, pos_hbm, xt_hbm,
                       rows_v, rows64_v, rows16_v, idx_v, idx64_v, idx16_v,
                       sem):
    c = lax.axis_index("c")
    s = lax.axis_index("s")
    w = c * NS + s

    base = w * ROWS_PER_TILE

    @pl.when(w < NW - 1)
    def _full_tile():
        for k in range(2):
            off = base + k * EB
            pltpu.sync_copy(y_hbm.at[pl.ds(off, EB)], rows_v)
            pltpu.sync_copy(pos_hbm.at[pl.ds(off, EB)], idx_v)
            pltpu.async_copy(rows_v, xt_hbm.at[idx_v], sem).wait()
        off = base + 2 * EB
        pltpu.sync_copy(y_hbm.at[pl.ds(off, 64)], rows64_v)
        pltpu.sync_copy(pos_hbm.at[pl.ds(off, 64)], idx64_v)
        pltpu.async_copy(rows64_v, xt_hbm.at[idx64_v], sem).wait()

    @pl.when(w == NW - 1)
    def _last_tile():
        pltpu.sync_copy(y_hbm.at[pl.ds(base, 64)], rows64_v)
        pltpu.sync_copy(pos_hbm.at[pl.ds(base, 64)], idx64_v)
        pltpu.async_copy(rows64_v, xt_hbm.at[idx64_v], sem).wait()
        pltpu.sync_copy(y_hbm.at[pl.ds(base + 64, 16)], rows16_v)
        pltpu.sync_copy(pos_hbm.at[pl.ds(base + 64, 16)], idx16_v)
        pltpu.async_copy(rows16_v, xt_hbm.at[idx16_v], sem).wait()


def _xws_body(xt_ref, hist_ref, w_ref, xws_ref, dinv_ref):
    deg = jnp.sum(hist_ref[...].astype(jnp.float32), axis=1, keepdims=True) + 1.0
    dinv = lax.rsqrt(deg)
    z = jnp.dot(xt_ref[...], w_ref[...], preferred_element_type=jnp.float32)
    xws_ref[...] = dinv * z
    dinv_ref[...] = dinv


def _run_xws(xt, hist2, conv_W):
    blk = 1000
    return pl.pallas_call(
        _xws_body,
        grid=(N // blk,),
        in_specs=[pl.BlockSpec((blk, 128), lambda i: (i, 0)),
                  pl.BlockSpec((blk, NW), lambda i: (i, 0)),
                  pl.BlockSpec((128, 128), lambda i: (0, 0))],
        out_specs=[pl.BlockSpec((blk, 128), lambda i: (i, 0)),
                   pl.BlockSpec((blk, 1), lambda i: (i, 0))],
        out_shape=[jax.ShapeDtypeStruct((N, 128), jnp.float32),
                   jax.ShapeDtypeStruct((N, 1), jnp.float32)],
    )(xt, hist2, conv_W)


EB3 = 104
NBLK3 = EDGES_PER_TILE // EB3
BODIES = NBLK3 // 6


@functools.partial(
    pl.kernel,
    mesh=_mesh,
    out_type=jax.ShapeDtypeStruct((NC, NPAD, 128), jnp.float32),
    scratch_types=[
        [[pltpu.VMEM((EB3,), jnp.int32) for _ in range(2)] for _ in range(3)],
        [[pltpu.VMEM((EB3,), jnp.int32) for _ in range(2)] for _ in range(3)],
        pltpu.VMEM((3, EB3, 128), jnp.float32),
        pltpu.VMEM((TAIL,), jnp.int32),
        pltpu.VMEM((TAIL,), jnp.int32),
        pltpu.VMEM((TAIL, 128), jnp.float32),
        pltpu.VMEM_SHARED((NPAD, 128), jnp.float32),
        [[pltpu.SemaphoreType.DMA for _ in range(2)] for _ in range(3)],
        [pltpu.SemaphoreType.DMA for _ in range(3)],
        [pltpu.SemaphoreType.DMA for _ in range(3)],
    ],
    compiler_params=_sc_params,
)
def _edge_pass_kernel(xws_hbm, src_hbm, dst_hbm, part_hbm,
                      sidx, didx, rows_v, sidx_t, didx_t, rows_t,
                      acc_sh, isem, gsem, ssem):
    c = lax.axis_index("c")
    s = lax.axis_index("s")
    w = c * NS + s

    def zero_row(i, _):
        r = i // 8
        k = i % 8
        rows_v[0, r, pl.ds(k * 16, 16)] = jnp.zeros((16,), jnp.float32)
        return _
    lax.fori_loop(0, EB3 * 8, zero_row, 0)
    rbase = s * HIST_PER_TILE
    for k in range(HIST_PER_TILE // EB3):
        pltpu.sync_copy(rows_v.at[0], acc_sh.at[pl.ds(rbase + k * EB3, EB3)])
    pltpu.sync_copy(rows_v.at[0, pl.ds(0, HIST_PER_TILE % EB3)],
                    acc_sh.at[pl.ds(rbase + (HIST_PER_TILE // EB3) * EB3,
                                    HIST_PER_TILE % EB3)])
    plsc.subcore_barrier()

    ebase = w * EDGES_PER_TILE

    def stage_idx(blk, slot, par):
        off = ebase + blk * EB3
        pltpu.async_copy(src_hbm.at[pl.ds(off, EB3)], sidx[slot][par],
                         isem[slot][par])
        pltpu.async_copy(dst_hbm.at[pl.ds(off, EB3)], didx[slot][par],
                         isem[slot][par])

    def iwait(slot, par):
        pltpu.make_async_copy(src_hbm.at[pl.ds(0, EB3)], sidx[slot][par],
                              isem[slot][par]).wait()
        pltpu.make_async_copy(dst_hbm.at[pl.ds(0, EB3)], didx[slot][par],
                              isem[slot][par]).wait()

    def swait(slot):
        pltpu.make_async_copy(xws_hbm.at[pl.ds(0, EB3)], rows_v.at[slot],
                              ssem[slot]).wait()

    def fire_scatter(slot, par):
        pltpu.async_copy(rows_v.at[slot], acc_sh.at[didx[slot][par]],
                         ssem[slot], add=True)

    for slot in range(3):
        stage_idx(slot, slot, 0)

    def body(i, _):
        g = [None] * 3
        for j in range(6):
            slot = j % 3
            par = 0 if j < 3 else 1
            if j < 3:
                @pl.when(i > 0)
                def _drain(slot=slot):
                    swait(slot)
            else:
                swait(slot)
            stage_idx(jnp.minimum(6 * i + j + 3, NBLK3 - 1), slot, 1 - par)
            iwait(slot, par)
            g[slot] = pltpu.async_copy(xws_hbm.at[sidx[slot][par]],
                                       rows_v.at[slot], gsem[slot])
            if j > 0:
                prev = (j - 1) % 3
                g[prev].wait()
                fire_scatter(prev, 0 if j - 1 < 3 else 1)
        g[2].wait()
        fire_scatter(2, 1)
        return _
    lax.fori_loop(0, BODIES, body, 0)

    for slot in range(3):
        swait(slot)
        iwait(slot, 0)

    toff = ebase + NBLK3 * EB3
    pltpu.sync_copy(src_hbm.at[pl.ds(toff, TAIL)], sidx_t)
    pltpu.sync_copy(dst_hbm.at[pl.ds(toff, TAIL)], didx_t)
    pltpu.async_copy(xws_hbm.at[sidx_t], rows_t, gsem[0]).wait()
    pltpu.async_copy(rows_t, acc_sh.at[didx_t], ssem[0], add=True).wait()

    plsc.subcore_barrier()

    pltpu.sync_copy(acc_sh.at[pl.ds(rbase, HIST_PER_TILE)],
                    part_hbm.at[c, pl.ds(rbase, HIST_PER_TILE)])


def _mid_body(p0_ref, p1_ref, xws_ref, dinv_ref, pr_ref, m_ref, b_ref, w_ref,
              out_ref):
    agg = p0_ref[...] + p1_ref[...] + xws_ref[...]
    dinv = dinv_ref[...]
    h = jnp.maximum(dinv * agg + b_ref[...], 0.0)
    h = jnp.where(m_ref[...] > 0, h, pr_ref[...])
    out_ref[...] = dinv * jnp.dot(h, w_ref[...],
                                  preferred_element_type=jnp.float32)


def _run_mid(p0, p1, xws, dinv, proj, mask_col, conv_b, next_W):
    blk = 1000
    row = lambda w: pl.BlockSpec((blk, w), lambda i: (i, 0))
    return pl.pallas_call(
        _mid_body,
        grid=(N // blk,),
        in_specs=[row(128), row(128), row(128), row(1), row(128), row(1),
                  pl.BlockSpec((1, 128), lambda i: (0, 0)),
                  pl.BlockSpec((128, 128), lambda i: (0, 0))],
        out_specs=row(128),
        out_shape=jax.ShapeDtypeStruct((N, 128), jnp.float32),
    )(p0, p1, xws, dinv, proj, mask_col, conv_b.reshape(1, -1), next_W)


def _final_body(q0_ref, q1_ref, xws_ref, dinv_ref, pr_ref, m_ref, b_ref,
                fcw_ref, fcb_ref, out_ref):
    agg = q0_ref[...] + q1_ref[...] + xws_ref[...]
    h = jnp.maximum(dinv_ref[...] * agg + b_ref[...], 0.0)
    h = jnp.where(m_ref[...] > 0, h, pr_ref[...])
    out_ref[...] = jnp.dot(h, fcw_ref[...],
                           preferred_element_type=jnp.float32) + fcb_ref[...]


def _run_final(q0, q1, xws, dinv, proj, mask_col, conv_b, fc_W, fc_b):
    blk = 1000
    row = lambda w: pl.BlockSpec((blk, w), lambda i: (i, 0))
    return pl.pallas_call(
        _final_body,
        grid=(N // blk,),
        in_specs=[row(128), row(128), row(128), row(1), row(128), row(1),
                  pl.BlockSpec((1, 128), lambda i: (0, 0)),
                  pl.BlockSpec((128, 64), lambda i: (0, 0)),
                  pl.BlockSpec((1, 64), lambda i: (0, 0))],
        out_specs=row(64),
        out_shape=jax.ShapeDtypeStruct((N, 64), jnp.float32),
    )(q0, q1, xws, dinv, proj, mask_col, conv_b.reshape(1, -1), fc_W,
      fc_b.reshape(1, -1))


def kernel(x, edge_index, mask, dnn_W1, dnn_b1, dnn_W2, dnn_b2,
           conv1_W, conv1_b, conv2_W, conv2_b, p1_W, p1_b, p2_W, p2_b,
           fc_W, fc_b):
    mask = mask.astype(bool)
    mask_col = mask.astype(jnp.int32).reshape(N, 1)
    mask_f = jnp.pad(mask.astype(jnp.float32), (0, NPAD - N)).reshape(80, 128)
    src = edge_index[0].astype(jnp.int32)
    dst = edge_index[1].astype(jnp.int32)

    histp = _hist_kernel(dst)
    y, proj1, proj2, pos = _run_dnn(x, mask_col, mask_f, dnn_W1, dnn_b1,
                                    dnn_W2, dnn_b2, p1_W, p1_b, p2_W, p2_b)
    xt = _xt_scatter_kernel(y, pos.reshape(NPAD))
    hist2 = histp.transpose(1, 0)

    xws1, dinv = _run_xws(xt, hist2, conv1_W)
    p = _edge_pass_kernel(xws1, src, dst)
    xws2 = _run_mid(p[0], p[1], xws1, dinv, proj1, mask_col, conv1_b, conv2_W)
    q = _edge_pass_kernel(xws2, src, dst)
    out = _run_final(q[0], q[1], xws2, dinv, proj2, mask_col, conv2_b,
                     fc_W, fc_b)
    return out
